# baseline (speedup 1.0000x reference)
import functools

import jax
import jax.numpy as jnp
from jax import lax
from jax.experimental import pallas as pl
from jax.experimental.pallas import tpu as pltpu
from jax.experimental.pallas import tpu_sc as plsc

_B = 16384
_D = 64
_NC = 2
_NS = 16
_L = 16
_NW = _NC * _NS
_BPW = _B // _NW
_GROUPS = _BPW // _L


def _rsqrt(v):
    i = lax.bitcast_convert_type(v, jnp.int32)
    i = jnp.int32(0x5F3759DF) - lax.shift_right_arithmetic(i, 1)
    y = lax.bitcast_convert_type(i, jnp.float32)
    vh = v * 0.5
    for _ in range(3):
        y = y * (1.5 - vh * y * y)
    return y


def _make_kernel():
    mesh = plsc.VectorSubcoreMesh(core_axis_name="c", subcore_axis_name="s")

    @functools.partial(
        pl.kernel,
        mesh=mesh,
        compiler_params=pltpu.CompilerParams(
            needs_layout_passes=False, use_tc_tiling_on_sc=False
        ),
        out_type=jax.ShapeDtypeStruct((_NW, _L), jnp.float32),
        scratch_types=[
            pltpu.VMEM((_BPW,), jnp.int32),
            pltpu.VMEM((_BPW, _D), jnp.float32),
            pltpu.VMEM((_BPW, _D), jnp.float32),
            pltpu.VMEM((_L,), jnp.float32),
            pltpu.SemaphoreType.DMA,
        ],
    )
    def run(x_hbm, lab_hbm, cen_hbm, out_hbm, lab_v, x_v, c_v, acc_v, sem):
        wid = lax.axis_index("s") * _NC + lax.axis_index("c")
        base = wid * _BPW
        pltpu.sync_copy(lab_hbm.at[pl.ds(base, _BPW)], lab_v)
        gather = pltpu.async_copy(cen_hbm.at[lab_v], c_v, sem)
        pltpu.sync_copy(x_hbm.at[pl.ds(base, _BPW)], x_v)
        gather.wait()

        lanes = lax.iota(jnp.int32, 16)

        def group_body(g, tot):
            rows = g * _L + lanes
            z = jnp.zeros((_L,), jnp.float32)
            sx, sc, dt = z, z, z
            for d in range(_D):
                col = jnp.full((_L,), d, jnp.int32)
                xv = plsc.load_gather(x_v, [rows, col])
                cv = plsc.load_gather(c_v, [rows, col])
                sx = sx + xv * xv
                sc = sc + cv * cv
                dt = dt + xv * cv
            rx = jnp.minimum(_rsqrt(sx), 1e12)
            rc = jnp.minimum(_rsqrt(sc), 1e12)
            dist = sx * rx * rx + sc * rc * rc - 2.0 * dt * rx * rc
            dist = jnp.clip(dist, 1e-12, 1e12)
            return tot + dist

        tot = lax.fori_loop(0, _GROUPS, group_body, jnp.zeros((_L,), jnp.float32))
        acc_v[...] = tot * (1.0 / _B)
        pltpu.sync_copy(acc_v, out_hbm.at[wid])

    return run


def kernel(x, labels, idx, centers):
    del idx
    run = _make_kernel()
    part = run(x, labels.astype(jnp.int32), centers)
    return jnp.sum(part)
